# Initial kernel scaffold; baseline (speedup 1.0000x reference)
import jax
import jax.numpy as jnp
from jax.experimental import pallas as pl


def kernel(x, position_ids):
    raise NotImplementedError("write your pallas kernel here")


# speedup vs baseline: 3.7409x; 3.7409x over previous
import jax
import jax.numpy as jnp
from jax.experimental import pallas as pl

_B, _S = 2, 8192
_HALF, _ROT = 64, 128
_THETA = 1000000.0
_BLK = 1024
_N = _B * _S


def _rope_body(pos_ref, cos_ref, sin_ref):
    p = pos_ref[...].astype(jnp.float32)
    j = jax.lax.broadcasted_iota(jnp.int32, (1, _HALF), 1).astype(jnp.float32)
    inv_freq = 1.0 / (_THETA ** (2.0 * j / _ROT))
    f = p * inv_freq
    emb = jnp.concatenate([f, f], axis=-1)
    cos_ref[...] = jnp.cos(emb)
    sin_ref[...] = jnp.sin(emb)


def kernel(x, position_ids):
    pos = position_ids.reshape(_N, 1)
    cos, sin = pl.pallas_call(
        _rope_body,
        grid=(_N // _BLK,),
        in_specs=[pl.BlockSpec((_BLK, 1), lambda i: (i, 0))],
        out_specs=[pl.BlockSpec((_BLK, _ROT), lambda i: (i, 0))] * 2,
        out_shape=[jax.ShapeDtypeStruct((_N, _ROT), jnp.float32)] * 2,
    )(pos)
    dt = x.dtype
    return (cos.reshape(_B, _S, _ROT).astype(dt), sin.reshape(_B, _S, _ROT).astype(dt))
